# baseline (speedup 1.0000x reference)
import functools

import jax
import jax.numpy as jnp
from jax import lax
from jax.experimental import pallas as pl
from jax.experimental.pallas import tpu as pltpu
from jax.experimental.pallas import tpu_sc as plsc

N_BINS = 1000
HIDDEN = 64
MIN_VAL = -5.0
SCALE = 100.0
HP_LEN = 1024

_info = plsc.get_sparse_core_info()
_NC, _NS = _info.num_cores, _info.num_subcores
_NW = _NC * _NS

CHUNK = 128

_GDN = lax.GatherDimensionNumbers(
    offset_dims=(), collapsed_slice_dims=(0,), start_index_map=(0,)
)


def _lane_broadcast(vec, e):
    idx = jnp.full((16, 1), e, dtype=jnp.int32)
    return lax.gather(
        vec, idx, dimension_numbers=_GDN, slice_sizes=(1,),
        mode=lax.GatherScatterMode.PROMISE_IN_BOUNDS,
    )


def _make_sc_call(total):
    per_w = total // _NW
    n_chunks = per_w // CHUNK
    n_groups = n_chunks // 2

    mesh = plsc.VectorSubcoreMesh(core_axis_name="c", subcore_axis_name="s")

    @functools.partial(
        pl.kernel,
        mesh=mesh,
        compiler_params=pltpu.CompilerParams(
            needs_layout_passes=False, use_tc_tiling_on_sc=True
        ),
        out_type=jax.ShapeDtypeStruct((total, HIDDEN), jnp.float32),
        scratch_types=[
            pltpu.VMEM((2 * CHUNK,), jnp.float32),
            pltpu.VMEM((2, CHUNK, HIDDEN), jnp.float32),
            pltpu.VMEM((N_BINS * HIDDEN,), jnp.float32),
            pltpu.VMEM((HP_LEN,), jnp.float32),
            pltpu.SemaphoreType.DMA,
            pltpu.SemaphoreType.DMA,
        ],
    )
    def sc_kernel(x_hbm, hp_hbm, table_hbm, out_hbm, x_v, rows_v, tab_v, hp_v,
                  xsem, ssem):
        wid = lax.axis_index("s") * _NC + lax.axis_index("c")
        wbase = wid * per_w

        pltpu.sync_copy(hp_hbm, hp_v)
        pltpu.sync_copy(table_hbm, tab_v)
        pltpu.async_copy(
            x_hbm.at[pl.ds(pl.multiple_of(wbase, CHUNK), CHUNK)],
            x_v.at[pl.ds(0, CHUNK)],
            xsem,
        )

        lane = lax.iota(jnp.int32, 16)

        def ebase(t):
            return pl.multiple_of(wbase + t * CHUNK, CHUNK)

        def wait_store(b):
            pltpu.make_async_copy(
                rows_v.at[b], out_hbm.at[pl.ds(0, CHUNK)], ssem
            ).wait()

        def group_body(g, carry):
            for b in range(2):
                t = g * 2 + b

                pl.when(g >= 1)(functools.partial(wait_store, b))

                pltpu.make_async_copy(
                    x_hbm.at[pl.ds(0, CHUNK)], x_v.at[pl.ds(b * CHUNK, CHUNK)], xsem
                ).wait()

                def prefetch():
                    pltpu.async_copy(
                        x_hbm.at[pl.ds(ebase(t + 1), CHUNK)],
                        x_v.at[pl.ds((1 - b) * CHUNK, CHUNK)],
                        xsem,
                    )
                if b == 0:
                    prefetch()
                else:
                    pl.when(g < n_groups - 1)(prefetch)

                for blk in range(CHUNK // 16):
                    xv = x_v[pl.ds(b * CHUNK + blk * 16, 16)]
                    t0 = (xv - MIN_VAL) * SCALE
                    gi = jnp.clip(t0.astype(jnp.int32), 0, N_BINS - 1)
                    hi = plsc.load_gather(hp_v, [gi + 1])
                    lo = plsc.load_gather(hp_v, [gi])
                    gi = gi + jnp.where(xv > hi, 1, 0) - jnp.where(xv <= lo, 1, 0)
                    gofs = gi * HIDDEN
                    for e in range(16):
                        base = _lane_broadcast(gofs, e)
                        row = blk * 16 + e
                        for k in range(0, HIDDEN, 16):
                            rows_v[b, row, pl.ds(k, 16)] = plsc.load_gather(
                                tab_v, [base + (lane + k)]
                            )

                pltpu.async_copy(
                    rows_v.at[b], out_hbm.at[pl.ds(ebase(t), CHUNK)], ssem
                )
            return carry

        lax.fori_loop(0, n_groups, group_body, 0)

        wait_store(0)
        wait_store(1)

    return sc_kernel


def kernel(x, boundaries, table):
    n_rows, row_len = x.shape
    total = n_rows * row_len
    xf = x.reshape(total)
    hp = jnp.concatenate(
        [
            jnp.full((1,), -jnp.inf, jnp.float32),
            boundaries.astype(jnp.float32),
            jnp.full((HP_LEN - 1 - boundaries.shape[0],), jnp.inf, jnp.float32),
        ]
    )
    tab_flat = table.reshape(N_BINS * HIDDEN)
    out = _make_sc_call(total)(xf, hp, tab_flat)
    return out.reshape(n_rows, row_len, HIDDEN)
